# baseline (speedup 1.0000x reference)
import functools

import jax
import jax.numpy as jnp
from jax.experimental import pallas as pl
from jax.experimental.pallas import tpu as pltpu

_B = 5000
_NEG_INF = float("-inf")


def _seg_kernel(lev_ref, x_ref, Wproj_ref, W1a_ref, W1b_ref, bproj_ref,
                b1_ref, W2_ref, b2_ref, Wo_ref, bo_ref,
                xbf_ref, pmb_ref, smask_ref, Wcb_ref, dWcb_ref, bcb_ref,
                dbcb_ref, w2bd_ref, mob_ref, sums_s, cnt_s, Wc_s,
                *, nlev, nblocks):
    i = pl.program_id(0)
    lev = lev_ref[0]
    B = lev.shape[0]
    hid = W1a_ref.shape[1]
    l0 = lev_ref[0, 0, 0]
    l1 = lev_ref[0, B - 1, 0]

    @pl.when(i == 0)
    def _():
        sums_s[...] = jnp.zeros_like(sums_s)
        cnt_s[...] = jnp.zeros_like(cnt_s)

    x_bf = x_ref[...].astype(jnp.bfloat16)
    xbf_ref[...] = x_bf
    onehot = (lev == jax.lax.broadcasted_iota(jnp.int32, (B, nlev), 1))
    oh_bf = onehot.astype(jnp.bfloat16)
    sums_s[...] += jax.lax.dot_general(
        oh_bf, x_bf, (((0,), (0,)), ((), ())),
        preferred_element_type=jnp.float32)

    @pl.when(l0 == l1)
    def _():
        cnt_s[...] += (
            jax.lax.broadcasted_iota(jnp.int32, cnt_s.shape, 1) == l0
        ).astype(jnp.float32) * float(B)

    @pl.when(l0 != l1)
    def _():
        cnt_s[...] += jnp.sum(onehot.astype(jnp.float32), axis=0,
                              keepdims=True)

    @pl.when(i == nblocks - 1)
    def _():
        for l in range(nlev):
            Wc_s[l] = jax.lax.dot_general(
                Wproj_ref[l], W1a_ref[...], (((1,), (0,)), ((), ())),
                preferred_element_type=jnp.float32)
        Wcb_ref[...] = Wc_s[...].astype(jnp.bfloat16)
        dWcb_ref[0] = Wcb_ref[0]
        for l in range(1, nlev):
            dWcb_ref[l] = (Wc_s[l] - Wc_s[l - 1]).astype(jnp.bfloat16)
        bc = jax.lax.dot_general(
            bproj_ref[...], W1a_ref[...], (((1,), (0,)), ((), ())),
            preferred_element_type=jnp.float32)
        bcb_ref[...] = bc
        dbcb_ref[...] = (bc - jnp.concatenate(
            [jnp.zeros_like(bc[0:1]), bc[:-1]], axis=0)).astype(jnp.bfloat16)

        rows = []
        for l in range(nlev):
            c = cnt_s[0, l]
            row = jax.lax.dot_general(
                sums_s[l:l + 1] * (1.0 / jnp.maximum(c, 1.0)),
                Wproj_ref[l], (((1,), (0,)), ((), ())),
                preferred_element_type=jnp.float32) + bproj_ref[l:l + 1]
            rows.append(jnp.where(c > 0.0, row, jnp.zeros_like(row)))
        means_h = jnp.concatenate(rows, axis=0)
        part_m = jax.lax.dot_general(
            means_h, W1b_ref[...], (((1,), (0,)), ((), ())),
            preferred_element_type=jnp.float32) + b1_ref[...]
        pmb_ref[...] = part_m.astype(jnp.bfloat16)
        mob_ref[...] = (jax.lax.dot_general(
            means_h, Wo_ref[...], (((1,), (0,)), ((), ())),
            preferred_element_type=jnp.float32)
            + bo_ref[...]).astype(jnp.bfloat16)

        smask_ref[...] = jnp.where(cnt_s[...] > 0.0,
                                   b2_ref[0, 0], _NEG_INF)

        w2rep = jnp.concatenate([W2_ref[...]] * nlev, axis=0)
        blk = jax.lax.broadcasted_iota(jnp.int32, (nlev * hid, nlev), 0) // hid
        col = jax.lax.broadcasted_iota(jnp.int32, (nlev * hid, nlev), 1)
        w2bd_ref[...] = jnp.where(blk == col, w2rep, 0.0).astype(jnp.bfloat16)


def _attn_kernel(lev_ref, xbf_ref, pmb_ref, smask_ref, Wcb_ref, dWcb_ref,
                 bcb_ref, dbcb_ref, w2bd_ref, mob_ref, gamma_ref,
                 beta_ref, out_ref, ph_ref, *, nlev):
    lev = lev_ref[0]
    B = lev.shape[0]
    x_bf = xbf_ref[...]
    l0 = lev_ref[0, 0, 0]
    l1 = lev_ref[0, B - 1, 0]

    W = Wcb_ref[pl.ds(l0, 1)].reshape(Wcb_ref.shape[1], Wcb_ref.shape[2])
    ph_ref[...] = jax.lax.dot_general(
        x_bf, W, (((1,), (0,)), ((), ())),
        preferred_element_type=jnp.float32).astype(jnp.bfloat16)
    for l in range(1, nlev):
        @pl.when((l0 < l) & (l <= l1))
        def _(l=l):
            mb = (lev >= l).astype(jnp.bfloat16)
            ph_ref[...] += jax.lax.dot_general(
                x_bf * mb, dWcb_ref[l], (((1,), (0,)), ((), ())),
                preferred_element_type=jnp.float32).astype(jnp.bfloat16
            ) + mb * dbcb_ref[l:l + 1]

    ph = ph_ref[...]
    pmx = pmb_ref[...] + bcb_ref[pl.ds(l0, 1)].astype(jnp.bfloat16)

    dall = jnp.concatenate(
        [jnp.tanh(ph + pmx[l:l + 1]) for l in range(nlev)],
        axis=1)
    scores = jax.lax.dot_general(
        dall, w2bd_ref[...], (((1,), (0,)), ((), ())),
        preferred_element_type=jnp.float32) + smask_ref[...]
    smax = jnp.max(scores, axis=1, keepdims=True)
    e = jnp.exp(scores - smax)
    wts = (e / jnp.sum(e, axis=1, keepdims=True)).astype(jnp.bfloat16)

    out = jax.lax.dot_general(wts, mob_ref[...], (((1,), (0,)), ((), ())),
                              preferred_element_type=jnp.float32)

    hid = out.shape[1]
    onesc = jnp.full((hid, 1), 1.0 / hid, dtype=jnp.float32)
    mu = jax.lax.dot_general(out, onesc, (((1,), (0,)), ((), ())),
                             preferred_element_type=jnp.float32)
    m2 = jax.lax.dot_general(out * out, onesc, (((1,), (0,)), ((), ())),
                             preferred_element_type=jnp.float32)
    inv = jax.lax.rsqrt(jnp.maximum(m2 - mu * mu, 0.0) + 1e-5)
    out = (out - mu) * inv
    out = out * gamma_ref[...] + beta_ref[...]
    out_ref[...] = jnp.maximum(out, 0.0)


def kernel(node_features, node_levels, W_proj, b_proj, W1, b1, W2, b2,
           Wo, bo, gamma, beta):
    N, IN = node_features.shape
    L, _, HID = W_proj.shape
    OUT = Wo.shape[1]
    B = _B
    NB = N // B
    assert NB * B == N

    lev3 = node_levels.reshape(NB, B, 1).astype(jnp.int32)
    W1a = W1[:HID]
    W1b = W1[HID:]

    full = lambda shape: pl.BlockSpec(shape, lambda i, _s=len(shape): (0,) * _s)

    (xbf, pmb, smask, Wcb, dWcb, bcb, dbcb, w2bd, mob) = pl.pallas_call(
        functools.partial(_seg_kernel, nlev=L, nblocks=NB),
        grid=(NB,),
        in_specs=[
            pl.BlockSpec((1, B, 1), lambda i: (i, 0, 0)),
            pl.BlockSpec((B, IN), lambda i: (i, 0)),
            full((L, IN, HID)),
            full((HID, HID)),
            full((HID, HID)),
            full((L, HID)),
            full((1, HID)),
            full((HID, 1)),
            full((1, 1)),
            full((HID, OUT)),
            full((1, OUT)),
        ],
        out_specs=[
            pl.BlockSpec((B, IN), lambda i: (i, 0)),
            full((L, HID)),
            full((1, L)),
            full((L, IN, HID)),
            full((L, IN, HID)),
            full((L, HID)),
            full((L, HID)),
            full((L * HID, L)),
            full((L, OUT)),
        ],
        out_shape=[
            jax.ShapeDtypeStruct((N, IN), jnp.bfloat16),
            jax.ShapeDtypeStruct((L, HID), jnp.bfloat16),
            jax.ShapeDtypeStruct((1, L), jnp.float32),
            jax.ShapeDtypeStruct((L, IN, HID), jnp.bfloat16),
            jax.ShapeDtypeStruct((L, IN, HID), jnp.bfloat16),
            jax.ShapeDtypeStruct((L, HID), jnp.float32),
            jax.ShapeDtypeStruct((L, HID), jnp.bfloat16),
            jax.ShapeDtypeStruct((L * HID, L), jnp.bfloat16),
            jax.ShapeDtypeStruct((L, OUT), jnp.bfloat16),
        ],
        scratch_shapes=[pltpu.VMEM((L, IN), jnp.float32),
                        pltpu.VMEM((1, L), jnp.float32),
                        pltpu.VMEM((L, IN, HID), jnp.float32)],
        compiler_params=pltpu.CompilerParams(
            dimension_semantics=("arbitrary",)),
    )(lev3, node_features, W_proj, W1a, W1b, b_proj, b1.reshape(1, HID),
      W2, b2.reshape(1, 1), Wo, bo.reshape(1, OUT))

    out = pl.pallas_call(
        functools.partial(_attn_kernel, nlev=L),
        grid=(NB,),
        in_specs=[
            pl.BlockSpec((1, B, 1), lambda i: (i, 0, 0)),
            pl.BlockSpec((B, IN), lambda i: (i, 0)),
            full((L, HID)),
            full((1, L)),
            full((L, IN, HID)),
            full((L, IN, HID)),
            full((L, HID)),
            full((L, HID)),
            full((L * HID, L)),
            full((L, OUT)),
            full((1, OUT)),
            full((1, OUT)),
        ],
        out_specs=pl.BlockSpec((B, OUT), lambda i: (i, 0)),
        out_shape=jax.ShapeDtypeStruct((N, OUT), jnp.float32),
        scratch_shapes=[pltpu.VMEM((B, HID), jnp.bfloat16)],
        compiler_params=pltpu.CompilerParams(
            dimension_semantics=("parallel",)),
    )(lev3, xbf, pmb, smask, Wcb, dWcb, bcb, dbcb, w2bd, mob,
      gamma.reshape(1, OUT), beta.reshape(1, OUT))

    return out


# speedup vs baseline: 1.2177x; 1.2177x over previous
import functools

import jax
import jax.numpy as jnp
from jax import lax
from jax.experimental import pallas as pl
from jax.experimental.pallas import tpu as pltpu
from jax.experimental.pallas import tpu_sc as plsc

_B = 5000
_NEG_INF = float("-inf")

_NC = 2
_NS = 16
_NW = _NC * _NS


def _sc_seg_build(N, IN, L, per, csize, nchunk, ntailw, tail_each):
    mesh = plsc.VectorSubcoreMesh(core_axis_name="c", subcore_axis_name="s")

    @functools.partial(
        pl.kernel,
        out_type=jax.ShapeDtypeStruct((_NC, L, IN), jnp.float32),
        mesh=mesh,
        scratch_types=[
            pltpu.VMEM((csize, IN), jnp.float32),
            pltpu.VMEM((csize,), jnp.int32),
            pltpu.VMEM((tail_each, IN), jnp.float32),
            pltpu.VMEM((tail_each,), jnp.int32),
            pltpu.VMEM_SHARED((L, IN), jnp.float32),
        ],
    )
    def sc_seg(x_hbm, lev_hbm, zs_hbm, sums_hbm,
               rows_v, idx_v, trows_v, tidx_v, sums_sh):
        c = lax.axis_index("c")
        s = lax.axis_index("s")
        wid = s * _NC + c

        @pl.when(s == 0)
        def _():
            pltpu.sync_copy(zs_hbm, sums_sh)
        plsc.subcore_barrier()

        base = wid * per
        for k in range(nchunk):
            off = base + k * csize
            pltpu.sync_copy(lev_hbm.at[pl.ds(off, csize)], idx_v)
            pltpu.sync_copy(x_hbm.at[pl.ds(off, csize)], rows_v)
            pltpu.sync_copy(rows_v, sums_sh.at[idx_v], add=True)

        @pl.when(wid < ntailw)
        def _():
            toff = _NW * per + wid * tail_each
            pltpu.sync_copy(lev_hbm.at[pl.ds(toff, tail_each)], tidx_v)
            pltpu.sync_copy(x_hbm.at[pl.ds(toff, tail_each)], trows_v)
            pltpu.sync_copy(trows_v, sums_sh.at[tidx_v], add=True)

        plsc.subcore_barrier()

        @pl.when(s == 0)
        def _():
            pltpu.sync_copy(sums_sh, sums_hbm.at[c])

    return sc_seg


def _prep_kernel(sumsp_ref, lev_ref, Wproj_ref, W1a_ref, W1b_ref,
                 bproj_ref, b1_ref, W2_ref, Wo_ref, bo_ref,
                 pmb_ref, cntrow_ref, Wcb_ref, dWcb_ref, bc_ref, dbc_ref,
                 w2bd_ref, mo_ref, Wc_s, *, nlev):
    hid = W1a_ref.shape[1]
    sums = sumsp_ref[0] + sumsp_ref[1]

    nb, bb = lev_ref.shape[0], lev_ref.shape[1]
    cnt_row = jnp.zeros((1, nlev), jnp.float32)
    for i in range(nb):
        oh = (lev_ref[i] == jax.lax.broadcasted_iota(jnp.int32, (bb, nlev), 1))
        cnt_row = cnt_row + jnp.sum(oh.astype(jnp.float32), axis=0,
                                    keepdims=True)
    cntrow_ref[...] = cnt_row

    for l in range(nlev):
        Wc_s[l] = jax.lax.dot_general(
            Wproj_ref[l], W1a_ref[...], (((1,), (0,)), ((), ())),
            preferred_element_type=jnp.float32)
    Wcb_ref[...] = Wc_s[...].astype(jnp.bfloat16)
    dWcb_ref[0] = Wcb_ref[0]
    for l in range(1, nlev):
        dWcb_ref[l] = (Wc_s[l] - Wc_s[l - 1]).astype(jnp.bfloat16)
    bc = jax.lax.dot_general(
        bproj_ref[...], W1a_ref[...], (((1,), (0,)), ((), ())),
        preferred_element_type=jnp.float32)
    bc_ref[...] = bc
    dbc_ref[...] = bc - jnp.concatenate(
        [jnp.zeros_like(bc[0:1]), bc[:-1]], axis=0)

    rows = []
    for l in range(nlev):
        cl = cntrow_ref[0, l]
        row = jax.lax.dot_general(
            sums[l:l + 1] * (1.0 / jnp.maximum(cl, 1.0)),
            Wproj_ref[l], (((1,), (0,)), ((), ())),
            preferred_element_type=jnp.float32) + bproj_ref[l:l + 1]
        rows.append(jnp.where(cl > 0.0, row, jnp.zeros_like(row)))
    means_h = jnp.concatenate(rows, axis=0)
    part_m = jax.lax.dot_general(
        means_h, W1b_ref[...], (((1,), (0,)), ((), ())),
        preferred_element_type=jnp.float32) + b1_ref[...]
    pmb_ref[...] = part_m.astype(jnp.bfloat16)
    mo_ref[...] = jax.lax.dot_general(
        means_h, Wo_ref[...], (((1,), (0,)), ((), ())),
        preferred_element_type=jnp.float32) + bo_ref[...]

    w2rep = jnp.concatenate([W2_ref[...]] * nlev, axis=0)
    blk = jax.lax.broadcasted_iota(jnp.int32, (nlev * hid, nlev), 0) // hid
    col = jax.lax.broadcasted_iota(jnp.int32, (nlev * hid, nlev), 1)
    w2bd_ref[...] = jnp.where(blk == col, w2rep, 0.0).astype(jnp.bfloat16)


def _attn_kernel(lev_ref, x_ref, pmb_ref, cntrow_ref, Wcb_ref, dWcb_ref,
                 bc_ref, dbc_ref, w2bd_ref, b2_ref, mo_ref, gamma_ref,
                 beta_ref, out_ref, ph_ref, *, nlev):
    lev = lev_ref[0]
    B = lev.shape[0]
    x_bf = x_ref[...].astype(jnp.bfloat16)
    l0 = lev_ref[0, 0, 0]
    l1 = lev_ref[0, B - 1, 0]

    W = Wcb_ref[pl.ds(l0, 1)].reshape(Wcb_ref.shape[1], Wcb_ref.shape[2])
    ph_ref[...] = jax.lax.dot_general(
        x_bf, W, (((1,), (0,)), ((), ())),
        preferred_element_type=jnp.float32) + bc_ref[pl.ds(l0, 1)]
    for l in range(1, nlev):
        @pl.when((l0 < l) & (l <= l1))
        def _(l=l):
            m = lev >= l
            ph_ref[...] += jax.lax.dot_general(
                jnp.where(m, x_bf, 0.0), dWcb_ref[l], (((1,), (0,)), ((), ())),
                preferred_element_type=jnp.float32
            ) + jnp.where(m, dbc_ref[l:l + 1], 0.0)

    ph_bf = ph_ref[...].astype(jnp.bfloat16)
    pmb = pmb_ref[...]

    dall = jnp.concatenate(
        [jnp.tanh(ph_bf + pmb[l:l + 1]) for l in range(nlev)],
        axis=1)
    scores = jax.lax.dot_general(
        dall, w2bd_ref[...], (((1,), (0,)), ((), ())),
        preferred_element_type=jnp.float32) + b2_ref[0, 0]
    valid = cntrow_ref[...] > 0.0
    scores = jnp.where(valid, scores, _NEG_INF)
    smax = jnp.max(scores, axis=1, keepdims=True)
    e = jnp.where(valid, jnp.exp(scores - smax), 0.0)
    wts = e / jnp.sum(e, axis=1, keepdims=True)

    out = jax.lax.dot_general(wts, mo_ref[...], (((1,), (0,)), ((), ())),
                              preferred_element_type=jnp.float32)
    mu = jnp.mean(out, axis=1, keepdims=True)
    var = jnp.mean((out - mu) * (out - mu), axis=1, keepdims=True)
    out = (out - mu) * jax.lax.rsqrt(var + 1e-5)
    out = out * gamma_ref[...] + beta_ref[...]
    out_ref[...] = jnp.maximum(out, 0.0)


def kernel(node_features, node_levels, W_proj, b_proj, W1, b1, W2, b2,
           Wo, bo, gamma, beta):
    N, IN = node_features.shape
    L, _, HID = W_proj.shape
    OUT = Wo.shape[1]
    B = _B
    NB = N // B
    assert NB * B == N

    lev3 = node_levels.reshape(NB, B, 1).astype(jnp.int32)
    lev1 = node_levels.astype(jnp.int32)
    W1a = W1[:HID]
    W1b = W1[HID:]

    per = (N // _NW) // 8 * 8
    nchunk = 5
    assert per % nchunk == 0
    csize = per // nchunk
    assert csize % 8 == 0
    tail = N - _NW * per
    tail_each = 8
    ntailw = tail // tail_each
    assert ntailw * tail_each == tail and ntailw <= _NW

    sc_seg = _sc_seg_build(N, IN, L, per, csize, nchunk, ntailw, tail_each)
    sums_p = sc_seg(node_features, lev1, jnp.zeros((L, IN), jnp.float32))

    full = lambda shape: pl.BlockSpec(shape, lambda i, _s=len(shape): (0,) * _s)

    (pmb, cnt_row, Wcb, dWcb, bc, dbc, w2bd, mo) = pl.pallas_call(
        functools.partial(_prep_kernel, nlev=L),
        grid=(1,),
        in_specs=[
            full((_NC, L, IN)),
            full((NB, B, 1)),
            full((L, IN, HID)),
            full((HID, HID)),
            full((HID, HID)),
            full((L, HID)),
            full((1, HID)),
            full((HID, 1)),
            full((HID, OUT)),
            full((1, OUT)),
        ],
        out_specs=[
            full((L, HID)),
            full((1, L)),
            full((L, IN, HID)),
            full((L, IN, HID)),
            full((L, HID)),
            full((L, HID)),
            full((L * HID, L)),
            full((L, OUT)),
        ],
        out_shape=[
            jax.ShapeDtypeStruct((L, HID), jnp.bfloat16),
            jax.ShapeDtypeStruct((1, L), jnp.float32),
            jax.ShapeDtypeStruct((L, IN, HID), jnp.bfloat16),
            jax.ShapeDtypeStruct((L, IN, HID), jnp.bfloat16),
            jax.ShapeDtypeStruct((L, HID), jnp.float32),
            jax.ShapeDtypeStruct((L, HID), jnp.float32),
            jax.ShapeDtypeStruct((L * HID, L), jnp.bfloat16),
            jax.ShapeDtypeStruct((L, OUT), jnp.float32),
        ],
        scratch_shapes=[pltpu.VMEM((L, IN, HID), jnp.float32)],
        compiler_params=pltpu.CompilerParams(
            dimension_semantics=("arbitrary",)),
    )(sums_p, lev3, W_proj, W1a, W1b, b_proj, b1.reshape(1, HID),
      W2, Wo, bo.reshape(1, OUT))

    out = pl.pallas_call(
        functools.partial(_attn_kernel, nlev=L),
        grid=(NB,),
        in_specs=[
            pl.BlockSpec((1, B, 1), lambda i: (i, 0, 0)),
            pl.BlockSpec((B, IN), lambda i: (i, 0)),
            full((L, HID)),
            full((1, L)),
            full((L, IN, HID)),
            full((L, IN, HID)),
            full((L, HID)),
            full((L, HID)),
            full((L * HID, L)),
            full((1, 1)),
            full((L, OUT)),
            full((1, OUT)),
            full((1, OUT)),
        ],
        out_specs=pl.BlockSpec((B, OUT), lambda i: (i, 0)),
        out_shape=jax.ShapeDtypeStruct((N, OUT), jnp.float32),
        scratch_shapes=[pltpu.VMEM((B, HID), jnp.float32)],
        compiler_params=pltpu.CompilerParams(
            dimension_semantics=("parallel",)),
    )(lev3, node_features, pmb, cnt_row, Wcb, dWcb, bc, dbc, w2bd,
      b2.reshape(1, 1), mo, gamma.reshape(1, OUT), beta.reshape(1, OUT))

    return out
